# Initial kernel scaffold; baseline (speedup 1.0000x reference)
import jax
import jax.numpy as jnp
from jax.experimental import pallas as pl


def kernel(x, Wa1, ba1, Wa2, ba2, W1, B1, W3, B3):
    raise NotImplementedError("write your pallas kernel here")


# speedup vs baseline: 8.5908x; 8.5908x over previous
import functools

import jax
import jax.numpy as jnp
import numpy as np
from jax import lax
from jax.experimental import pallas as pl
from jax.experimental.pallas import tpu as pltpu
from jax.experimental.pallas import tpu_sc as plsc

N = 10000
N_IN = 4000
N_FN = 5500
N_OUT = 500
E = 160000
C = 4
LAYERS = 4
B = 16
LATENT = 100
FC = N_FN * C

EH = E // 2
NP = 10240
EPAD = EH + 16
FCPAD = 22016
OUTPAD = 512
K = 2048

_rng = np.random.RandomState(0)
_SRC = _rng.randint(0, N_IN + N_FN, size=E).astype(np.int64)
_DST = _rng.randint(N_IN, N, size=E).astype(np.int64)

_e1 = np.nonzero(_DST < N_IN + N_FN)[0]
_f1 = _DST[_e1] - N_IN
_e3 = np.nonzero(_SRC >= N_IN)[0]
_f3 = _SRC[_e3] - N_IN
_eo = np.nonzero(_DST >= N_IN + N_FN)[0]
_do = _DST[_eo] - (N_IN + N_FN)

NNZ1 = _e1.size * C
NNZ3 = _e3.size * C

_i1 = int(np.searchsorted(_e1, EH))
_i3 = int(np.searchsorted(_e3, EH))
_io = int(np.searchsorted(_eo, EH))

S1 = (_i1 * C, (_e1.size - _i1) * C)
S3 = (_i3 * C, (_e3.size - _i3) * C)
SO = (_io, _eo.size - _io)

NCH1 = -(-max(S1) // K)
NCH3 = -(-max(S3) // K)
NCHO = -(-max(SO) // K)


def _pad_to(a, n, fill):
    out = np.full((n,), fill, dtype=np.int32)
    out[: a.size] = a.astype(np.int32)
    return out


def _build_streams():
    xi4 = np.zeros((2, NCH1, K), np.int32)
    tg1 = np.zeros((2, NCH1, K), np.int32)
    hx3 = np.zeros((2, NCH3, K), np.int32)
    ei3 = np.zeros((2, NCH3, K), np.int32)
    oei = np.zeros((2, NCHO, K), np.int32)
    otg = np.zeros((2, NCHO, K), np.int32)
    srch = np.zeros((2, EH), np.int32)
    for h in range(2):
        e1h = _e1[_i1:] - EH if h else _e1[:_i1]
        f1h = _f1[_i1:] if h else _f1[:_i1]
        e3h = _e3[_i3:] - EH if h else _e3[:_i3]
        f3h = _f3[_i3:] if h else _f3[:_i3]
        eoh = _eo[_io:] - EH if h else _eo[:_io]
        doh = _do[_io:] if h else _do[:_io]
        n1, n3 = e1h.size * C, e3h.size * C
        xi4[h] = _pad_to(np.repeat(e1h, C), NCH1 * K, EH).reshape(NCH1, K)
        tg1[h] = _pad_to(np.repeat(f1h, C) * C + np.tile(np.arange(C), e1h.size),
                         NCH1 * K, FC).reshape(NCH1, K)
        hx3[h] = _pad_to(np.repeat(f3h, C) * C + np.tile(np.arange(C), e3h.size),
                         NCH3 * K, FC).reshape(NCH3, K)
        ei3[h] = _pad_to(np.repeat(e3h, C), NCH3 * K, EH).reshape(NCH3, K)
        oei[h] = _pad_to(eoh, NCHO * K, EH).reshape(NCHO, K)
        otg[h] = _pad_to(doh, NCHO * K, OUTPAD - 1).reshape(NCHO, K)
        srch[h] = _SRC[h * EH:(h + 1) * EH].astype(np.int32)
    return xi4, tg1, hx3, ei3, oei, otg, srch


_XI4, _TG1, _HX3, _EI3, _OEI, _OTG, _SRCH = (jnp.asarray(a) for a in _build_streams())


def _tc_body(x_ref, wa1_ref, ba1_ref, wa2_ref, ba2_ref, s_ref, xz_ref):
    x = x_ref[...]
    xom = x[:, :N_IN]
    h = xom @ wa1_ref[...] + ba1_ref[...]
    h = jnp.where(h > 0, h, jnp.exp(jnp.minimum(h, 0.0)) - 1.0)
    o = h @ wa2_ref[...] + ba2_ref[...]
    mu = jnp.mean(o, axis=1, keepdims=True)
    var = jnp.mean((o - mu) ** 2, axis=1, keepdims=True)
    s = jax.nn.sigmoid((o - mu) * lax.rsqrt(var + 1e-5))
    s_ref[...] = jnp.pad(s, ((0, 0), (0, FCPAD - FC)))
    mask = lax.broadcasted_iota(jnp.int32, (B, N), 1) < N_IN
    xz_ref[...] = jnp.pad(jnp.where(mask, 0.0, x), ((0, 0), (0, NP - N)))


def _tc_state(x2d, Wa1, ba1, Wa2, ba2):
    return pl.pallas_call(
        _tc_body,
        out_shape=[
            jax.ShapeDtypeStruct((B, FCPAD), jnp.float32),
            jax.ShapeDtypeStruct((B, NP), jnp.float32),
        ],
    )(x2d, Wa1, ba1.reshape(1, LATENT), Wa2, ba2.reshape(1, FC))


def _rsqrt(v):
    i = plsc.bitcast(v, jnp.int32)
    i = 0x5F3759DF - lax.shift_right_logical(i, 1)
    y = plsc.bitcast(i, jnp.float32)
    for _ in range(3):
        y = y * (1.5 - 0.5 * v * y * y)
    return y


def _elu(x):
    return jnp.where(x > 0, x, jnp.exp(jnp.minimum(x, 0.0)) - 1.0)


def _vloop(n_vregs, unroll, body):
    assert n_vregs % unroll == 0

    def outer(i, carry):
        for u in range(unroll):
            body(i * unroll + u)
        return carry

    lax.fori_loop(0, n_vregs // unroll, outer, 0)


def _sc_body(xz, spad, w1p, b1p, w3p, b3, srch, xi4, tg1, hx3, ei3, oei, otg,
             out, xe_v, h_v, ia_v, ib_v, wa_v, wb_v, ob_v, hs_sp, os_sp):
    cid = lax.axis_index("c")
    sid = lax.axis_index("s")
    p = sid // 2
    b = cid * 8 + p
    hf = sid % 2
    zero16 = jnp.zeros((16,), jnp.float32)
    i4 = lax.iota(jnp.int32, 16) * 4

    pltpu.sync_copy(xz.at[b], h_v.at[pl.ds(0, NP)])

    def xe0_chunk(c, clen, base):
        pltpu.sync_copy(srch.at[hf, pl.ds(base, clen)], ia_v.at[pl.ds(0, clen)])

        def one(i):
            idx = ia_v[pl.ds(i * 16, 16)]
            xe_v[pl.ds(base + i * 16, 16)] = plsc.load_gather(h_v, [idx])

        _vloop(clen // 16, 4, one)

    def xe0_loop(c, carry):
        xe0_chunk(c, 2048, c * 2048)
        return carry

    lax.fori_loop(0, 39, xe0_loop, 0)
    xe0_chunk(39, 128, 39 * 2048)
    xe_v[pl.ds(EH, 16)] = zero16

    def layer(l, carry):
        def zh(i):
            h_v[pl.ds(i * 16, 16)] = zero16

        _vloop(FCPAD // 16, 8, zh)

        def w1_chunk(c, carry):
            pltpu.sync_copy(xi4.at[hf, c], ia_v.at[pl.ds(0, K)])
            pltpu.sync_copy(tg1.at[hf, c], ib_v.at[pl.ds(0, K)])
            pltpu.sync_copy(w1p.at[l, hf, c], wa_v.at[pl.ds(0, K)])

            def one(i):
                xi = ia_v[pl.ds(i * 16, 16)]
                tg = ib_v[pl.ds(i * 16, 16)]
                w = wa_v[pl.ds(i * 16, 16)]
                xv = plsc.load_gather(xe_v, [xi])
                plsc.addupdate_scatter(h_v, [tg], xv * w)

            _vloop(K // 16, 4, one)
            return carry

        lax.fori_loop(0, NCH1, w1_chunk, 0)

        @pl.when(hf == 0)
        def _():
            pltpu.sync_copy(h_v, hs_sp.at[p])

        plsc.subcore_barrier()

        @pl.when(hf == 1)
        def _():
            def merge_chunk(c, clen, base):
                pltpu.sync_copy(hs_sp.at[p, pl.ds(base, clen)],
                                wa_v.at[pl.ds(0, clen)])
                pltpu.sync_copy(b1p.at[l, pl.ds(base, clen)],
                                wb_v.at[pl.ds(0, clen)])

                def one(i):
                    d = pl.ds(i * 16, 16)
                    bb = pl.ds(base + i * 16, 16)
                    h_v[bb] = h_v[bb] + wa_v[d] + wb_v[d]

                _vloop(clen // 16, 4, one)

            def merge_loop(c, carry):
                merge_chunk(c, 2048, c * 2048)
                return carry

            lax.fori_loop(0, 10, merge_loop, 0)
            merge_chunk(10, 1536, 10 * 2048)
            pltpu.sync_copy(h_v, hs_sp.at[p])

        plsc.subcore_barrier()

        @pl.when(hf == 0)
        def _():
            pltpu.sync_copy(hs_sp.at[p], h_v)

        def norm_chunk(cs, clen, base):
            pltpu.sync_copy(spad.at[b, pl.ds(base, clen)], wa_v.at[pl.ds(0, clen)])

            def blk(bi, carry):
                o = bi * 64
                hidx = [i4 + (base + o + c0) for c0 in range(C)]
                ch = [plsc.load_gather(h_v, [hidx[c0]]) for c0 in range(C)]
                mu = (ch[0] + ch[1] + ch[2] + ch[3]) * 0.25
                d = [ch[c0] - mu for c0 in range(C)]
                var = (d[0] * d[0] + d[1] * d[1] + d[2] * d[2] + d[3] * d[3]) * 0.25
                r = _rsqrt(var + 1e-5)
                for c0 in range(C):
                    sv = plsc.load_gather(wa_v, [i4 + (o + c0)])
                    res = _elu(sv * (d[c0] * r))
                    plsc.store_scatter(h_v, [hidx[c0]], res)
                return carry

            lax.fori_loop(0, clen // 64, blk, 0)

        def norm_loop(cs, carry):
            norm_chunk(cs, 2048, cs * 2048)
            return carry

        lax.fori_loop(0, 10, norm_loop, 0)
        norm_chunk(10, 1536, 10 * 2048)

        def w3_chunk(c, carry):
            pltpu.sync_copy(hx3.at[hf, c], ia_v.at[pl.ds(0, K)])
            pltpu.sync_copy(ei3.at[hf, c], ib_v.at[pl.ds(0, K)])
            pltpu.sync_copy(w3p.at[l, hf, c], wa_v.at[pl.ds(0, K)])

            def one(i):
                hx = ia_v[pl.ds(i * 16, 16)]
                ei = ib_v[pl.ds(i * 16, 16)]
                w = wa_v[pl.ds(i * 16, 16)]
                hv = plsc.load_gather(h_v, [hx])
                plsc.addupdate_scatter(xe_v, [ei], hv * w)

            _vloop(K // 16, 4, one)
            return carry

        lax.fori_loop(0, NCH3, w3_chunk, 0)

        def b3_chunk(c, clen, base):
            pltpu.sync_copy(b3.at[l, pl.ds(hf * EH + base, clen)],
                            wa_v.at[pl.ds(0, clen)])

            def one(i):
                dd = pl.ds(base + i * 16, 16)
                xe_v[dd] = xe_v[dd] + wa_v[pl.ds(i * 16, 16)]

            _vloop(clen // 16, 4, one)

        def b3_loop(c, carry):
            b3_chunk(c, 2048, c * 2048)
            return carry

        lax.fori_loop(0, 39, b3_loop, 0)
        b3_chunk(39, 128, 39 * 2048)
        return carry

    lax.fori_loop(0, LAYERS, layer, 0)

    def zo(i):
        ob_v[pl.ds(i * 16, 16)] = zero16

    _vloop(OUTPAD // 16, 4, zo)

    def out_chunk(c, carry):
        pltpu.sync_copy(oei.at[hf, c], ia_v.at[pl.ds(0, K)])
        pltpu.sync_copy(otg.at[hf, c], ib_v.at[pl.ds(0, K)])

        def one(i):
            xi = ia_v[pl.ds(i * 16, 16)]
            tg = ib_v[pl.ds(i * 16, 16)]
            xv = plsc.load_gather(xe_v, [xi])
            plsc.addupdate_scatter(ob_v, [tg], xv * (1.0 / LAYERS))

        _vloop(K // 16, 4, one)
        return carry

    lax.fori_loop(0, NCHO, out_chunk, 0)

    @pl.when(hf == 1)
    def _():
        pltpu.sync_copy(ob_v, os_sp.at[p])

    plsc.subcore_barrier()

    @pl.when(hf == 0)
    def _():
        pltpu.sync_copy(os_sp.at[p], wa_v.at[pl.ds(0, OUTPAD)])

        def one(i):
            d = pl.ds(i * 16, 16)
            ob_v[d] = ob_v[d] + wa_v[d]

        _vloop(OUTPAD // 16, 4, one)
        pltpu.sync_copy(ob_v, out.at[b])


@functools.cache
def _get_sc_kernel():
    return functools.partial(
        pl.kernel,
        out_type=jax.ShapeDtypeStruct((B, OUTPAD), jnp.float32),
        mesh=plsc.VectorSubcoreMesh(core_axis_name="c", subcore_axis_name="s"),
        compiler_params=pltpu.CompilerParams(needs_layout_passes=False),
        scratch_types=[
            pltpu.VMEM((EPAD,), jnp.float32),
            pltpu.VMEM((FCPAD,), jnp.float32),
            pltpu.VMEM((2048,), jnp.int32),
            pltpu.VMEM((2048,), jnp.int32),
            pltpu.VMEM((2048,), jnp.float32),
            pltpu.VMEM((2048,), jnp.float32),
            pltpu.VMEM((OUTPAD,), jnp.float32),
            pltpu.VMEM_SHARED((8, FCPAD), jnp.float32),
            pltpu.VMEM_SHARED((8, OUTPAD), jnp.float32),
        ],
    )(_sc_body)


def _pad_halves(w, s0, s1, nch):
    h0 = jnp.pad(w[:, : s0], ((0, 0), (0, nch * K - s0)))
    h1 = jnp.pad(w[:, s0:], ((0, 0), (0, nch * K - s1)))
    return jnp.stack([h0, h1], axis=1).reshape(LAYERS, 2, nch, K)


@jax.jit
def kernel(x, Wa1, ba1, Wa2, ba2, W1, B1, W3, B3):
    spad, xz = _tc_state(x[:, :, 0], Wa1, ba1, Wa2, ba2)
    w1p = _pad_halves(W1, S1[0], S1[1], NCH1)
    w3p = _pad_halves(W3, S3[0], S3[1], NCH3)
    b1p = jnp.pad(B1, ((0, 0), (0, FCPAD - FC)))
    out = _get_sc_kernel()(xz, spad, w1p, b1p, w3p, B3,
                           _SRCH, _XI4, _TG1, _HX3, _EI3, _OEI, _OTG)
    return out[:, :N_OUT]


# speedup vs baseline: 12.0182x; 1.3990x over previous
import functools

import jax
import jax.numpy as jnp
import numpy as np
from jax import lax
from jax.experimental import pallas as pl
from jax.experimental.pallas import tpu as pltpu
from jax.experimental.pallas import tpu_sc as plsc

N = 10000
N_IN = 4000
N_FN = 5500
N_OUT = 500
E = 160000
C = 4
LAYERS = 4
B = 16
LATENT = 100
FC = N_FN * C

EH = E // 2
NP = 10240
EPAD = EH + 16
FCPAD = 22016
OUTPAD = 512
KE = 1024
PKW = 6 * KE
STG = 8192

_rng = np.random.RandomState(0)
_SRC = _rng.randint(0, N_IN + N_FN, size=E).astype(np.int64)
_DST = _rng.randint(N_IN, N, size=E).astype(np.int64)

_e1 = np.nonzero(_DST < N_IN + N_FN)[0]
_f1 = _DST[_e1] - N_IN
_e3 = np.nonzero(_SRC >= N_IN)[0]
_f3 = _SRC[_e3] - N_IN
_eo = np.nonzero(_DST >= N_IN + N_FN)[0]
_do = _DST[_eo] - (N_IN + N_FN)

NNZ1 = _e1.size * C
NNZ3 = _e3.size * C

_i1 = int(np.searchsorted(_e1, EH))
_i3 = int(np.searchsorted(_e3, EH))
_io = int(np.searchsorted(_eo, EH))

S1 = (_i1, _e1.size - _i1)
S3 = (_i3, _e3.size - _i3)
SO = (_io, _eo.size - _io)

NCH1 = 2 * (-(-max(S1) // (2 * KE)))
NCH3 = 2 * (-(-max(S3) // (2 * KE)))
KO = 2048
NCHO = -(-max(SO) // KO)


def _pad_to(a, n, fill):
    out = np.full((n,), fill, dtype=np.int32)
    out[: a.size] = a.astype(np.int32)
    return out


def _build_streams():
    spk1 = np.zeros((2, NCH1, 2 * KE), np.int32)
    spk3 = np.zeros((2, NCH3, 2 * KE), np.int32)
    pout = np.zeros((2, NCHO, 2 * KO), np.int32)
    srch = np.zeros((2, EH), np.int32)
    for h in range(2):
        e1h = _e1[_i1:] - EH if h else _e1[:_i1]
        f1h = _f1[_i1:] if h else _f1[:_i1]
        e3h = _e3[_i3:] - EH if h else _e3[:_i3]
        f3h = _f3[_i3:] if h else _f3[:_i3]
        eoh = _eo[_io:] - EH if h else _eo[:_io]
        doh = _do[_io:] if h else _do[:_io]
        xi = _pad_to(e1h, NCH1 * KE, EH).reshape(NCH1, KE)
        tg = _pad_to(f1h * C, NCH1 * KE, FC).reshape(NCH1, KE)
        spk1[h] = np.concatenate([xi, tg], axis=1)
        hb = _pad_to(f3h * C, NCH3 * KE, FC).reshape(NCH3, KE)
        ei = _pad_to(e3h, NCH3 * KE, EH).reshape(NCH3, KE)
        spk3[h] = np.concatenate([hb, ei], axis=1)
        oi = _pad_to(eoh, NCHO * KO, EH).reshape(NCHO, KO)
        ot = _pad_to(doh, NCHO * KO, OUTPAD - 1).reshape(NCHO, KO)
        pout[h] = np.concatenate([oi, ot], axis=1)
        srch[h] = _SRC[h * EH:(h + 1) * EH].astype(np.int32)
    return (spk1.view(np.float32), spk3.view(np.float32),
            pout.view(np.float32), srch.view(np.float32))


_SPK1, _SPK3, _POUT, _SRCH = (jnp.asarray(a) for a in _build_streams())


def _tc_body(x_ref, wa1_ref, ba1_ref, wa2_ref, ba2_ref, s_ref, xz_ref):
    x = x_ref[...]
    xom = x[:, :N_IN]
    h = xom @ wa1_ref[...] + ba1_ref[...]
    h = jnp.where(h > 0, h, jnp.exp(jnp.minimum(h, 0.0)) - 1.0)
    o = h @ wa2_ref[...] + ba2_ref[...]
    mu = jnp.mean(o, axis=1, keepdims=True)
    var = jnp.mean((o - mu) ** 2, axis=1, keepdims=True)
    s = jax.nn.sigmoid((o - mu) * lax.rsqrt(var + 1e-5))
    s_ref[...] = jnp.pad(s, ((0, 0), (0, FCPAD - FC)))
    mask = lax.broadcasted_iota(jnp.int32, (B, N), 1) < N_IN
    xz_ref[...] = jnp.pad(jnp.where(mask, 0.0, x), ((0, 0), (0, NP - N)))


def _tc_state(x2d, Wa1, ba1, Wa2, ba2):
    return pl.pallas_call(
        _tc_body,
        out_shape=[
            jax.ShapeDtypeStruct((B, FCPAD), jnp.float32),
            jax.ShapeDtypeStruct((B, NP), jnp.float32),
        ],
    )(x2d, Wa1, ba1.reshape(1, LATENT), Wa2, ba2.reshape(1, FC))


def _rsqrt(v):
    i = plsc.bitcast(v, jnp.int32)
    i = 0x5F3759DF - lax.shift_right_logical(i, 1)
    y = plsc.bitcast(i, jnp.float32)
    for _ in range(3):
        y = y * (1.5 - 0.5 * v * y * y)
    return y


def _elu(x):
    return jnp.where(x > 0, x, jnp.exp(jnp.minimum(x, 0.0)) - 1.0)


def _vloop(n_vregs, unroll, body):
    assert n_vregs % unroll == 0

    def outer(i, carry):
        for u in range(unroll):
            body(i * unroll + u)
        return carry

    lax.fori_loop(0, n_vregs // unroll, outer, 0)


def _i32(v):
    return plsc.bitcast(v, jnp.int32)


def _sc_body(xz, spad, pk1, b1p, pk3, b3, srch, pout,
             out, xe_v, h_v, pk_v, ob_v, hs_sp, os_sp, sem0, sem1):
    cid = lax.axis_index("c")
    sid = lax.axis_index("s")
    p = sid // 2
    b = cid * 8 + p
    hf = sid % 2
    zero16 = jnp.zeros((16,), jnp.float32)
    i4 = lax.iota(jnp.int32, 16) * 4

    bufs = (pk_v.at[pl.ds(0, PKW)], pk_v.at[pl.ds(PKW, PKW)])
    sems = (sem0, sem1)

    def ring(src3d, nch, compute):
        def step(c, carry):
            pltpu.sync_copy(src3d.at[hf, c], bufs[0])
            compute(0)
            return carry

        lax.fori_loop(0, nch, step, 0)

    pltpu.sync_copy(xz.at[b], h_v.at[pl.ds(0, NP)])

    def xe0_chunk(clen, base):
        pltpu.sync_copy(srch.at[hf, pl.ds(base, clen)], pk_v.at[pl.ds(0, clen)])

        def one(i):
            idx = _i32(pk_v[pl.ds(i * 16, 16)])
            xe_v[pl.ds(base + i * 16, 16)] = plsc.load_gather(h_v, [idx])

        _vloop(clen // 16, 4, one)

    def xe0_loop(c, carry):
        xe0_chunk(STG, c * STG)
        return carry

    lax.fori_loop(0, 9, xe0_loop, 0)
    xe0_chunk(6272, 9 * STG)
    xe_v[pl.ds(EH, 16)] = zero16

    def layer(l, carry):
        def zh(i):
            h_v[pl.ds(i * 16, 16)] = zero16

        _vloop(FCPAD // 16, 8, zh)

        def w1_compute(u):
            base = u * PKW

            def one(j):
                d = j * 16
                xi = _i32(pk_v[pl.ds(base + d, 16)])
                tg = _i32(pk_v[pl.ds(base + KE + d, 16)])
                xv = plsc.load_gather(xe_v, [xi])
                for c0 in range(C):
                    w = pk_v[pl.ds(base + 2 * KE + c0 * KE + d, 16)]
                    plsc.addupdate_scatter(h_v, [tg + c0], xv * w)

            _vloop(KE // 16, 4, one)

        ring(pk1.at[l], NCH1, w1_compute)

        @pl.when(hf == 0)
        def _():
            pltpu.sync_copy(h_v, hs_sp.at[p])

        plsc.subcore_barrier()

        @pl.when(hf == 1)
        def _():
            def merge_chunk(clen, base):
                pltpu.sync_copy(hs_sp.at[p, pl.ds(base, clen)],
                                pk_v.at[pl.ds(0, clen)])
                pltpu.sync_copy(b1p.at[l, pl.ds(base, clen)],
                                pk_v.at[pl.ds(4096, clen)])

                def one(i):
                    d = i * 16
                    bb = pl.ds(base + d, 16)
                    h_v[bb] = (h_v[bb] + pk_v[pl.ds(d, 16)]
                               + pk_v[pl.ds(4096 + d, 16)])

                _vloop(clen // 16, 4, one)

            def merge_loop(c, carry):
                merge_chunk(4096, c * 4096)
                return carry

            lax.fori_loop(0, 5, merge_loop, 0)
            merge_chunk(1536, 5 * 4096)
            pltpu.sync_copy(h_v, hs_sp.at[p])

        plsc.subcore_barrier()

        @pl.when(hf == 0)
        def _():
            pltpu.sync_copy(hs_sp.at[p], h_v)

        def norm_chunk(clen, base):
            pltpu.sync_copy(spad.at[b, pl.ds(base, clen)], pk_v.at[pl.ds(0, clen)])

            def blk(bi, carry):
                o = bi * 64
                hidx = [i4 + (base + o + c0) for c0 in range(C)]
                ch = [plsc.load_gather(h_v, [hidx[c0]]) for c0 in range(C)]
                mu = (ch[0] + ch[1] + ch[2] + ch[3]) * 0.25
                d = [ch[c0] - mu for c0 in range(C)]
                var = (d[0] * d[0] + d[1] * d[1] + d[2] * d[2] + d[3] * d[3]) * 0.25
                r = _rsqrt(var + 1e-5)
                for c0 in range(C):
                    sv = plsc.load_gather(pk_v, [i4 + (o + c0)])
                    res = _elu(sv * (d[c0] * r))
                    plsc.store_scatter(h_v, [hidx[c0]], res)
                return carry

            lax.fori_loop(0, clen // 64, blk, 0)

        def norm_loop(cs, carry):
            norm_chunk(STG, cs * STG)
            return carry

        lax.fori_loop(0, 2, norm_loop, 0)
        norm_chunk(5632, 2 * STG)

        def w3_compute(u):
            base = u * PKW

            def one(j):
                d = j * 16
                hb = _i32(pk_v[pl.ds(base + d, 16)])
                ei = _i32(pk_v[pl.ds(base + KE + d, 16)])
                acc = zero16
                for c0 in range(C):
                    w = pk_v[pl.ds(base + 2 * KE + c0 * KE + d, 16)]
                    acc = acc + plsc.load_gather(h_v, [hb + c0]) * w
                plsc.addupdate_scatter(xe_v, [ei], acc)

            _vloop(KE // 16, 4, one)

        ring(pk3.at[l], NCH3, w3_compute)

        def b3_chunk(clen, base):
            pltpu.sync_copy(b3.at[l, pl.ds(hf * EH + base, clen)],
                            pk_v.at[pl.ds(0, clen)])

            def one(i):
                dd = pl.ds(base + i * 16, 16)
                xe_v[dd] = xe_v[dd] + pk_v[pl.ds(i * 16, 16)]

            _vloop(clen // 16, 4, one)

        def b3_loop(c, carry):
            b3_chunk(STG, c * STG)
            return carry

        lax.fori_loop(0, 9, b3_loop, 0)
        b3_chunk(6272, 9 * STG)
        return carry

    lax.fori_loop(0, LAYERS, layer, 0)

    def zo(i):
        ob_v[pl.ds(i * 16, 16)] = zero16

    _vloop(OUTPAD // 16, 4, zo)

    def out_chunk(c, carry):
        pltpu.sync_copy(pout.at[hf, c], pk_v.at[pl.ds(0, 2 * KO)])

        def one(i):
            d = i * 16
            xi = _i32(pk_v[pl.ds(d, 16)])
            tg = _i32(pk_v[pl.ds(KO + d, 16)])
            xv = plsc.load_gather(xe_v, [xi])
            plsc.addupdate_scatter(ob_v, [tg], xv * (1.0 / LAYERS))

        _vloop(KO // 16, 4, one)
        return carry

    lax.fori_loop(0, NCHO, out_chunk, 0)

    @pl.when(hf == 1)
    def _():
        pltpu.sync_copy(ob_v, os_sp.at[p])

    plsc.subcore_barrier()

    @pl.when(hf == 0)
    def _():
        pltpu.sync_copy(os_sp.at[p], pk_v.at[pl.ds(0, OUTPAD)])

        def one(i):
            d = pl.ds(i * 16, 16)
            ob_v[d] = ob_v[d] + pk_v[d]

        _vloop(OUTPAD // 16, 4, one)
        pltpu.sync_copy(ob_v, out.at[b])


@functools.cache
def _get_sc_kernel():
    return functools.partial(
        pl.kernel,
        out_type=jax.ShapeDtypeStruct((B, OUTPAD), jnp.float32),
        mesh=plsc.VectorSubcoreMesh(core_axis_name="c", subcore_axis_name="s"),
        compiler_params=pltpu.CompilerParams(needs_layout_passes=False),
        scratch_types=[
            pltpu.VMEM((EPAD,), jnp.float32),
            pltpu.VMEM((FCPAD,), jnp.float32),
            pltpu.VMEM((2 * PKW,), jnp.float32),
            pltpu.VMEM((OUTPAD,), jnp.float32),
            pltpu.VMEM_SHARED((8, FCPAD), jnp.float32),
            pltpu.VMEM_SHARED((8, OUTPAD), jnp.float32),
            pltpu.SemaphoreType.DMA,
            pltpu.SemaphoreType.DMA,
        ],
    )(_sc_body)


def _pack(w, s0, s1, nch, spk):
    h0 = jnp.pad(w[:, : s0 * C], ((0, 0), (0, nch * KE * C - s0 * C)))
    h1 = jnp.pad(w[:, s0 * C:], ((0, 0), (0, nch * KE * C - s1 * C)))
    wh = jnp.stack([h0, h1], axis=1).reshape(LAYERS, 2, nch, KE, C)
    wh = jnp.swapaxes(wh, -1, -2)
    wf = wh.reshape(LAYERS, 2, nch, 4 * KE)
    sb = jnp.broadcast_to(spk[None], (LAYERS, 2, nch, 2 * KE))
    return jnp.concatenate([sb, wf], axis=-1)


@jax.jit
def kernel(x, Wa1, ba1, Wa2, ba2, W1, B1, W3, B3):
    spad, xz = _tc_state(x[:, :, 0], Wa1, ba1, Wa2, ba2)
    pk1 = _pack(W1, S1[0], S1[1], NCH1, _SPK1)
    pk3 = _pack(W3, S3[0], S3[1], NCH3, _SPK3)
    b1p = jnp.pad(B1, ((0, 0), (0, FCPAD - FC)))
    out = _get_sc_kernel()(xz, spad, pk1, b1p, pk3, B3, _SRCH, _POUT)
    return out[:, :N_OUT]


# speedup vs baseline: 12.1578x; 1.0116x over previous
import functools

import jax
import jax.numpy as jnp
import numpy as np
from jax import lax
from jax.experimental import pallas as pl
from jax.experimental.pallas import tpu as pltpu
from jax.experimental.pallas import tpu_sc as plsc

N = 10000
N_IN = 4000
N_FN = 5500
N_OUT = 500
E = 160000
C = 4
LAYERS = 4
B = 16
LATENT = 100
FC = N_FN * C

EH = E // 2
NP = 10240
EPAD = EH + 16
FCPAD = 22016
OUTPAD = 512
KE = 2048
PKW = 6 * KE
STG = 8192

_rng = np.random.RandomState(0)
_SRC = _rng.randint(0, N_IN + N_FN, size=E).astype(np.int64)
_DST = _rng.randint(N_IN, N, size=E).astype(np.int64)

_e1 = np.nonzero(_DST < N_IN + N_FN)[0]
_f1 = _DST[_e1] - N_IN
_e3 = np.nonzero(_SRC >= N_IN)[0]
_f3 = _SRC[_e3] - N_IN
_eo = np.nonzero(_DST >= N_IN + N_FN)[0]
_do = _DST[_eo] - (N_IN + N_FN)

NNZ1 = _e1.size * C
NNZ3 = _e3.size * C

_i1 = int(np.searchsorted(_e1, EH))
_i3 = int(np.searchsorted(_e3, EH))
_io = int(np.searchsorted(_eo, EH))

S1 = (_i1, _e1.size - _i1)
S3 = (_i3, _e3.size - _i3)
SO = (_io, _eo.size - _io)

NCH1 = -(-max(S1) // KE)
NCH3 = -(-max(S3) // KE)
KO = 2048
NCHO = -(-max(SO) // KO)


def _pad_to(a, n, fill):
    out = np.full((n,), fill, dtype=np.int32)
    out[: a.size] = a.astype(np.int32)
    return out


def _build_streams():
    spk1 = np.zeros((2, NCH1, 2 * KE), np.int32)
    spk3 = np.zeros((2, NCH3, 2 * KE), np.int32)
    pout = np.zeros((2, NCHO, 2 * KO), np.int32)
    srch = np.zeros((2, EH), np.int32)
    for h in range(2):
        e1h = _e1[_i1:] - EH if h else _e1[:_i1]
        f1h = _f1[_i1:] if h else _f1[:_i1]
        e3h = _e3[_i3:] - EH if h else _e3[:_i3]
        f3h = _f3[_i3:] if h else _f3[:_i3]
        eoh = _eo[_io:] - EH if h else _eo[:_io]
        doh = _do[_io:] if h else _do[:_io]
        xi = _pad_to(e1h, NCH1 * KE, EH).reshape(NCH1, KE)
        tg = _pad_to(f1h * C, NCH1 * KE, FC).reshape(NCH1, KE)
        spk1[h] = np.concatenate([xi, tg], axis=1)
        hb = _pad_to(f3h * C, NCH3 * KE, FC).reshape(NCH3, KE)
        ei = _pad_to(e3h, NCH3 * KE, EH).reshape(NCH3, KE)
        spk3[h] = np.concatenate([hb, ei], axis=1)
        oi = _pad_to(eoh, NCHO * KO, EH).reshape(NCHO, KO)
        ot = _pad_to(doh, NCHO * KO, OUTPAD - 1).reshape(NCHO, KO)
        pout[h] = np.concatenate([oi, ot], axis=1)
        srch[h] = _SRC[h * EH:(h + 1) * EH].astype(np.int32)
    return (spk1.view(np.float32), spk3.view(np.float32),
            pout.view(np.float32), srch.view(np.float32))


_SPK1, _SPK3, _POUT, _SRCH = (jnp.asarray(a) for a in _build_streams())


def _tc_body(x_ref, wa1_ref, ba1_ref, wa2_ref, ba2_ref, s_ref, xz_ref):
    x = x_ref[...]
    xom = x[:, :N_IN]
    h = xom @ wa1_ref[...] + ba1_ref[...]
    h = jnp.where(h > 0, h, jnp.exp(jnp.minimum(h, 0.0)) - 1.0)
    o = h @ wa2_ref[...] + ba2_ref[...]
    mu = jnp.mean(o, axis=1, keepdims=True)
    var = jnp.mean((o - mu) ** 2, axis=1, keepdims=True)
    s = jax.nn.sigmoid((o - mu) * lax.rsqrt(var + 1e-5))
    s_ref[...] = jnp.pad(s, ((0, 0), (0, FCPAD - FC)))
    mask = lax.broadcasted_iota(jnp.int32, (B, N), 1) < N_IN
    xz_ref[...] = jnp.pad(jnp.where(mask, 0.0, x), ((0, 0), (0, NP - N)))


def _tc_state(x2d, Wa1, ba1, Wa2, ba2):
    return pl.pallas_call(
        _tc_body,
        out_shape=[
            jax.ShapeDtypeStruct((B, FCPAD), jnp.float32),
            jax.ShapeDtypeStruct((B, NP), jnp.float32),
        ],
    )(x2d, Wa1, ba1.reshape(1, LATENT), Wa2, ba2.reshape(1, FC))


def _rsqrt(v):
    i = plsc.bitcast(v, jnp.int32)
    i = 0x5F3759DF - lax.shift_right_logical(i, 1)
    y = plsc.bitcast(i, jnp.float32)
    for _ in range(3):
        y = y * (1.5 - 0.5 * v * y * y)
    return y


def _elu(x):
    return jnp.where(x > 0, x, jnp.exp(jnp.minimum(x, 0.0)) - 1.0)


def _vloop(n_vregs, unroll, body):
    assert n_vregs % unroll == 0

    def outer(i, carry):
        for u in range(unroll):
            body(i * unroll + u)
        return carry

    lax.fori_loop(0, n_vregs // unroll, outer, 0)


def _i32(v):
    return plsc.bitcast(v, jnp.int32)


_GDN = lax.GatherDimensionNumbers(
    offset_dims=(), collapsed_slice_dims=(0,), start_index_map=(0,))


def _perm(v, idx):
    return lax.gather(v, idx[:, None], _GDN, slice_sizes=(1,),
                      mode=lax.GatherScatterMode.PROMISE_IN_BOUNDS)


def _sc_body(xz, spad, pk1, b1p, pk3, b3, srch, pout,
             out, xe_v, h_v, pk_v, ob_v, hs_sp, os_sp):
    cid = lax.axis_index("c")
    sid = lax.axis_index("s")
    p = sid // 2
    b = cid * 8 + p
    hf = sid % 2
    zero16 = jnp.zeros((16,), jnp.float32)
    i4 = lax.iota(jnp.int32, 16) * 4

    p1 = lax.iota(jnp.int32, 16) ^ 1
    p2 = lax.iota(jnp.int32, 16) ^ 2

    def ring(src3d, nch, compute):
        def step(c, carry):
            pltpu.sync_copy(src3d.at[hf, c], pk_v.at[pl.ds(0, PKW)])
            compute(0)
            return carry

        lax.fori_loop(0, nch, step, 0)

    pltpu.sync_copy(xz.at[b], h_v.at[pl.ds(0, NP)])

    def xe0_chunk(clen, base):
        pltpu.sync_copy(srch.at[hf, pl.ds(base, clen)], pk_v.at[pl.ds(0, clen)])

        def one(i):
            idx = _i32(pk_v[pl.ds(i * 16, 16)])
            xe_v[pl.ds(base + i * 16, 16)] = plsc.load_gather(h_v, [idx])

        _vloop(clen // 16, 4, one)

    def xe0_loop(c, carry):
        xe0_chunk(STG, c * STG)
        return carry

    lax.fori_loop(0, 9, xe0_loop, 0)
    xe0_chunk(6272, 9 * STG)
    xe_v[pl.ds(EH, 16)] = zero16

    def layer(l, carry):
        def zh(i):
            h_v[pl.ds(i * 16, 16)] = zero16

        _vloop(FCPAD // 16, 8, zh)

        def w1_compute(u):
            base = u * PKW

            def one(j):
                d = j * 16
                xi = _i32(pk_v[pl.ds(base + d, 16)])
                tg = _i32(pk_v[pl.ds(base + KE + d, 16)])
                xv = plsc.load_gather(xe_v, [xi])
                for c0 in range(C):
                    w = pk_v[pl.ds(base + 2 * KE + c0 * KE + d, 16)]
                    plsc.addupdate_scatter(h_v, [tg + c0], xv * w)

            _vloop(KE // 16, 4, one)

        ring(pk1.at[l], NCH1, w1_compute)

        @pl.when(hf == 0)
        def _():
            pltpu.sync_copy(h_v, hs_sp.at[p])

        plsc.subcore_barrier()

        @pl.when(hf == 1)
        def _():
            def merge_chunk(clen, base):
                pltpu.sync_copy(hs_sp.at[p, pl.ds(base, clen)],
                                pk_v.at[pl.ds(0, clen)])
                pltpu.sync_copy(b1p.at[l, pl.ds(base, clen)],
                                pk_v.at[pl.ds(4096, clen)])

                def one(i):
                    d = i * 16
                    bb = pl.ds(base + d, 16)
                    h_v[bb] = (h_v[bb] + pk_v[pl.ds(d, 16)]
                               + pk_v[pl.ds(4096 + d, 16)])

                _vloop(clen // 16, 4, one)

            def merge_loop(c, carry):
                merge_chunk(4096, c * 4096)
                return carry

            lax.fori_loop(0, 5, merge_loop, 0)
            merge_chunk(1536, 5 * 4096)
            pltpu.sync_copy(h_v, hs_sp.at[p])

        plsc.subcore_barrier()

        @pl.when(hf == 0)
        def _():
            pltpu.sync_copy(hs_sp.at[p], h_v)

        def norm_chunk(clen, base):
            pltpu.sync_copy(spad.at[b, pl.ds(base, clen)], pk_v.at[pl.ds(0, clen)])

            def one(i):
                bb = pl.ds(base + i * 16, 16)
                v = h_v[bb]
                sv = pk_v[pl.ds(i * 16, 16)]
                t = v + _perm(v, p1)
                gs = t + _perm(t, p2)
                mu = gs * 0.25
                d = v - mu
                sq = d * d
                q = sq + _perm(sq, p1)
                var = (q + _perm(q, p2)) * 0.25
                r = _rsqrt(var + 1e-5)
                h_v[bb] = _elu(sv * (d * r))

            _vloop(clen // 16, 4, one)

        def norm_loop(cs, carry):
            norm_chunk(STG, cs * STG)
            return carry

        lax.fori_loop(0, 2, norm_loop, 0)
        norm_chunk(5632, 2 * STG)

        def w3_compute(u):
            base = u * PKW

            def one(j):
                d = j * 16
                hb = _i32(pk_v[pl.ds(base + d, 16)])
                ei = _i32(pk_v[pl.ds(base + KE + d, 16)])
                acc = zero16
                for c0 in range(C):
                    w = pk_v[pl.ds(base + 2 * KE + c0 * KE + d, 16)]
                    acc = acc + plsc.load_gather(h_v, [hb + c0]) * w
                plsc.addupdate_scatter(xe_v, [ei], acc)

            _vloop(KE // 16, 4, one)

        ring(pk3.at[l], NCH3, w3_compute)

        def b3_chunk(clen, base):
            pltpu.sync_copy(b3.at[l, pl.ds(hf * EH + base, clen)],
                            pk_v.at[pl.ds(0, clen)])

            def one(i):
                dd = pl.ds(base + i * 16, 16)
                xe_v[dd] = xe_v[dd] + pk_v[pl.ds(i * 16, 16)]

            _vloop(clen // 16, 4, one)

        def b3_loop(c, carry):
            b3_chunk(STG, c * STG)
            return carry

        lax.fori_loop(0, 9, b3_loop, 0)
        b3_chunk(6272, 9 * STG)
        return carry

    lax.fori_loop(0, LAYERS, layer, 0)

    def zo(i):
        ob_v[pl.ds(i * 16, 16)] = zero16

    _vloop(OUTPAD // 16, 4, zo)

    def out_chunk(c, carry):
        pltpu.sync_copy(pout.at[hf, c], pk_v.at[pl.ds(0, 2 * KO)])

        def one(i):
            d = i * 16
            xi = _i32(pk_v[pl.ds(d, 16)])
            tg = _i32(pk_v[pl.ds(KO + d, 16)])
            xv = plsc.load_gather(xe_v, [xi])
            plsc.addupdate_scatter(ob_v, [tg], xv * (1.0 / LAYERS))

        _vloop(KO // 16, 4, one)
        return carry

    lax.fori_loop(0, NCHO, out_chunk, 0)

    @pl.when(hf == 1)
    def _():
        pltpu.sync_copy(ob_v, os_sp.at[p])

    plsc.subcore_barrier()

    @pl.when(hf == 0)
    def _():
        pltpu.sync_copy(os_sp.at[p], pk_v.at[pl.ds(0, OUTPAD)])

        def one(i):
            d = pl.ds(i * 16, 16)
            ob_v[d] = ob_v[d] + pk_v[d]

        _vloop(OUTPAD // 16, 4, one)
        pltpu.sync_copy(ob_v, out.at[b])


@functools.cache
def _get_sc_kernel():
    return functools.partial(
        pl.kernel,
        out_type=jax.ShapeDtypeStruct((B, OUTPAD), jnp.float32),
        mesh=plsc.VectorSubcoreMesh(core_axis_name="c", subcore_axis_name="s"),
        compiler_params=pltpu.CompilerParams(needs_layout_passes=False),
        scratch_types=[
            pltpu.VMEM((EPAD,), jnp.float32),
            pltpu.VMEM((FCPAD,), jnp.float32),
            pltpu.VMEM((PKW,), jnp.float32),
            pltpu.VMEM((OUTPAD,), jnp.float32),
            pltpu.VMEM_SHARED((8, FCPAD), jnp.float32),
            pltpu.VMEM_SHARED((8, OUTPAD), jnp.float32),
        ],
    )(_sc_body)


def _pack(w, s0, s1, nch, spk):
    h0 = jnp.pad(w[:, : s0 * C], ((0, 0), (0, nch * KE * C - s0 * C)))
    h1 = jnp.pad(w[:, s0 * C:], ((0, 0), (0, nch * KE * C - s1 * C)))
    wh = jnp.stack([h0, h1], axis=1).reshape(LAYERS, 2, nch, KE, C)
    wh = jnp.swapaxes(wh, -1, -2)
    wf = wh.reshape(LAYERS, 2, nch, 4 * KE)
    sb = jnp.broadcast_to(spk[None], (LAYERS, 2, nch, 2 * KE))
    return jnp.concatenate([sb, wf], axis=-1)


@jax.jit
def kernel(x, Wa1, ba1, Wa2, ba2, W1, B1, W3, B3):
    spad, xz = _tc_state(x[:, :, 0], Wa1, ba1, Wa2, ba2)
    pk1 = _pack(W1, S1[0], S1[1], NCH1, _SPK1)
    pk3 = _pack(W3, S3[0], S3[1], NCH3, _SPK3)
    b1p = jnp.pad(B1, ((0, 0), (0, FCPAD - FC)))
    out = _get_sc_kernel()(xz, spad, pk1, b1p, pk3, B3, _SRCH, _POUT)
    return out[:, :N_OUT]


# speedup vs baseline: 13.4966x; 1.1101x over previous
import functools

import jax
import jax.numpy as jnp
import numpy as np
from jax import lax
from jax.experimental import pallas as pl
from jax.experimental.pallas import tpu as pltpu
from jax.experimental.pallas import tpu_sc as plsc

N = 10000
N_IN = 4000
N_FN = 5500
N_OUT = 500
E = 160000
C = 4
LAYERS = 4
B = 16
LATENT = 100
FC = N_FN * C

EH = E // 2
NP = 10240
EPAD = EH + 16
FCPAD = 22016
OUTPAD = 512
KE = 1024
PKW = 6 * KE
STG = 8192

_rng = np.random.RandomState(0)
_SRC = _rng.randint(0, N_IN + N_FN, size=E).astype(np.int64)
_DST = _rng.randint(N_IN, N, size=E).astype(np.int64)

_e1 = np.nonzero(_DST < N_IN + N_FN)[0]
_f1 = _DST[_e1] - N_IN
_e3 = np.nonzero(_SRC >= N_IN)[0]
_f3 = _SRC[_e3] - N_IN
_eo = np.nonzero(_DST >= N_IN + N_FN)[0]
_do = _DST[_eo] - (N_IN + N_FN)

NNZ1 = _e1.size * C
NNZ3 = _e3.size * C

_i1 = int(np.searchsorted(_e1, EH))
_i3 = int(np.searchsorted(_e3, EH))
_io = int(np.searchsorted(_eo, EH))

S1 = (_i1, _e1.size - _i1)
S3 = (_i3, _e3.size - _i3)
SO = (_io, _eo.size - _io)

NCH1 = 2 * (-(-max(S1) // (2 * KE)))
NCH3 = 2 * (-(-max(S3) // (2 * KE)))
KO = 2048
NCHO = -(-max(SO) // KO)


def _pad_to(a, n, fill):
    out = np.full((n,), fill, dtype=np.int32)
    out[: a.size] = a.astype(np.int32)
    return out


def _build_streams():
    spk1 = np.zeros((2, NCH1, 2 * KE), np.int32)
    spk3 = np.zeros((2, NCH3, 2 * KE), np.int32)
    pout = np.zeros((2, NCHO, 2 * KO), np.int32)
    srch = np.zeros((2, EH), np.int32)
    for h in range(2):
        e1h = _e1[_i1:] - EH if h else _e1[:_i1]
        f1h = _f1[_i1:] if h else _f1[:_i1]
        e3h = _e3[_i3:] - EH if h else _e3[:_i3]
        f3h = _f3[_i3:] if h else _f3[:_i3]
        eoh = _eo[_io:] - EH if h else _eo[:_io]
        doh = _do[_io:] if h else _do[:_io]
        xi = _pad_to(e1h, NCH1 * KE, EH).reshape(NCH1, KE)
        tg = _pad_to(f1h * C, NCH1 * KE, FC).reshape(NCH1, KE)
        spk1[h] = np.concatenate([xi, tg], axis=1)
        hb = _pad_to(f3h * C, NCH3 * KE, FC).reshape(NCH3, KE)
        ei = _pad_to(e3h, NCH3 * KE, EH).reshape(NCH3, KE)
        spk3[h] = np.concatenate([hb, ei], axis=1)
        oi = _pad_to(eoh, NCHO * KO, EH).reshape(NCHO, KO)
        ot = _pad_to(doh, NCHO * KO, OUTPAD - 1).reshape(NCHO, KO)
        pout[h] = np.concatenate([oi, ot], axis=1)
        srch[h] = _SRC[h * EH:(h + 1) * EH].astype(np.int32)
    return (spk1.view(np.float32), spk3.view(np.float32),
            pout.view(np.float32), srch.view(np.float32))


_SPK1, _SPK3, _POUT, _SRCH = (jnp.asarray(a) for a in _build_streams())


def _tc_body(x_ref, wa1_ref, ba1_ref, wa2_ref, ba2_ref, s_ref, xz_ref):
    x = x_ref[...]
    xom = x[:, :N_IN]
    h = xom @ wa1_ref[...] + ba1_ref[...]
    h = jnp.where(h > 0, h, jnp.exp(jnp.minimum(h, 0.0)) - 1.0)
    o = h @ wa2_ref[...] + ba2_ref[...]
    mu = jnp.mean(o, axis=1, keepdims=True)
    var = jnp.mean((o - mu) ** 2, axis=1, keepdims=True)
    s = jax.nn.sigmoid((o - mu) * lax.rsqrt(var + 1e-5))
    s_ref[...] = jnp.pad(s, ((0, 0), (0, FCPAD - FC)))
    mask = lax.broadcasted_iota(jnp.int32, (B, N), 1) < N_IN
    xz_ref[...] = jnp.pad(jnp.where(mask, 0.0, x), ((0, 0), (0, NP - N)))


def _tc_state(x2d, Wa1, ba1, Wa2, ba2):
    return pl.pallas_call(
        _tc_body,
        out_shape=[
            jax.ShapeDtypeStruct((B, FCPAD), jnp.float32),
            jax.ShapeDtypeStruct((B, NP), jnp.float32),
        ],
    )(x2d, Wa1, ba1.reshape(1, LATENT), Wa2, ba2.reshape(1, FC))


def _rsqrt(v):
    i = plsc.bitcast(v, jnp.int32)
    i = 0x5F3759DF - lax.shift_right_logical(i, 1)
    y = plsc.bitcast(i, jnp.float32)
    for _ in range(3):
        y = y * (1.5 - 0.5 * v * y * y)
    return y


def _elu(x):
    return jnp.where(x > 0, x, jnp.exp(jnp.minimum(x, 0.0)) - 1.0)


def _vloop(n_vregs, unroll, body):
    assert n_vregs % unroll == 0

    def outer(i, carry):
        for u in range(unroll):
            body(i * unroll + u)
        return carry

    lax.fori_loop(0, n_vregs // unroll, outer, 0)


def _i32(v):
    return plsc.bitcast(v, jnp.int32)


_GDN = lax.GatherDimensionNumbers(
    offset_dims=(), collapsed_slice_dims=(0,), start_index_map=(0,))


def _perm(v, idx):
    return lax.gather(v, idx[:, None], _GDN, slice_sizes=(1,),
                      mode=lax.GatherScatterMode.PROMISE_IN_BOUNDS)


def _sc_body(xz, spad, pk1, b1p, pk3, b3, srch, pout,
             out, xe_v, h_v, pk_v, ob_v, hs_sp, os_sp, sem0, sem1):
    cid = lax.axis_index("c")
    sid = lax.axis_index("s")
    p = sid // 2
    b = cid * 8 + p
    hf = sid % 2
    zero16 = jnp.zeros((16,), jnp.float32)
    i4 = lax.iota(jnp.int32, 16) * 4

    p1 = lax.iota(jnp.int32, 16) ^ 1
    p2 = lax.iota(jnp.int32, 16) ^ 2

    bufs = (pk_v.at[pl.ds(0, PKW)], pk_v.at[pl.ds(PKW, PKW)])
    sems = (sem0, sem1)

    def ring(src3d, nch, compute):
        pltpu.async_copy(src3d.at[hf, 0], bufs[0], sems[0])

        def step(i, carry):
            for u in range(2):
                c = i * 2 + u
                nxt = 1 - u

                @pl.when(c + 1 < nch)
                def _():
                    pltpu.async_copy(src3d.at[hf, c + 1], bufs[nxt], sems[nxt])

                pltpu.make_async_copy(src3d.at[hf, 0], bufs[u], sems[u]).wait()
                compute(u)
            return carry

        lax.fori_loop(0, nch // 2, step, 0)

    pltpu.sync_copy(xz.at[b], h_v.at[pl.ds(0, NP)])

    def xe0_chunk(clen, base):
        pltpu.sync_copy(srch.at[hf, pl.ds(base, clen)], pk_v.at[pl.ds(0, clen)])

        def one(i):
            idx = _i32(pk_v[pl.ds(i * 16, 16)])
            xe_v[pl.ds(base + i * 16, 16)] = plsc.load_gather(h_v, [idx])

        _vloop(clen // 16, 4, one)

    def xe0_loop(c, carry):
        xe0_chunk(STG, c * STG)
        return carry

    lax.fori_loop(0, 9, xe0_loop, 0)
    xe0_chunk(6272, 9 * STG)
    xe_v[pl.ds(EH, 16)] = zero16

    def layer(l, carry):
        def zh(i):
            h_v[pl.ds(i * 16, 16)] = zero16

        _vloop(FCPAD // 16, 8, zh)

        def w1_compute(u):
            base = u * PKW

            def one(j):
                d = j * 16
                xi = _i32(pk_v[pl.ds(base + d, 16)])
                tg = _i32(pk_v[pl.ds(base + KE + d, 16)])
                xv = plsc.load_gather(xe_v, [xi])
                for c0 in range(C):
                    w = pk_v[pl.ds(base + 2 * KE + c0 * KE + d, 16)]
                    plsc.addupdate_scatter(h_v, [tg + c0], xv * w)

            _vloop(KE // 16, 4, one)

        ring(pk1.at[l], NCH1, w1_compute)

        @pl.when(hf == 0)
        def _():
            pltpu.sync_copy(h_v, hs_sp.at[p])

        plsc.subcore_barrier()

        @pl.when(hf == 1)
        def _():
            def merge_chunk(clen, base):
                pltpu.sync_copy(hs_sp.at[p, pl.ds(base, clen)],
                                pk_v.at[pl.ds(0, clen)])
                pltpu.sync_copy(b1p.at[l, pl.ds(base, clen)],
                                pk_v.at[pl.ds(4096, clen)])

                def one(i):
                    d = i * 16
                    bb = pl.ds(base + d, 16)
                    h_v[bb] = (h_v[bb] + pk_v[pl.ds(d, 16)]
                               + pk_v[pl.ds(4096 + d, 16)])

                _vloop(clen // 16, 4, one)

            def merge_loop(c, carry):
                merge_chunk(4096, c * 4096)
                return carry

            lax.fori_loop(0, 5, merge_loop, 0)
            merge_chunk(1536, 5 * 4096)
            pltpu.sync_copy(h_v, hs_sp.at[p])

        plsc.subcore_barrier()

        @pl.when(hf == 0)
        def _():
            pltpu.sync_copy(hs_sp.at[p], h_v)

        def norm_chunk(clen, base):
            pltpu.sync_copy(spad.at[b, pl.ds(base, clen)], pk_v.at[pl.ds(0, clen)])

            def one(i):
                bb = pl.ds(base + i * 16, 16)
                v = h_v[bb]
                sv = pk_v[pl.ds(i * 16, 16)]
                t = v + _perm(v, p1)
                gs = t + _perm(t, p2)
                mu = gs * 0.25
                d = v - mu
                sq = d * d
                q = sq + _perm(sq, p1)
                var = (q + _perm(q, p2)) * 0.25
                r = _rsqrt(var + 1e-5)
                h_v[bb] = _elu(sv * (d * r))

            _vloop(clen // 16, 4, one)

        def norm_loop(cs, carry):
            norm_chunk(STG, cs * STG)
            return carry

        lax.fori_loop(0, 2, norm_loop, 0)
        norm_chunk(5632, 2 * STG)

        def w3_compute(u):
            base = u * PKW

            def one(j):
                d = j * 16
                hb = _i32(pk_v[pl.ds(base + d, 16)])
                ei = _i32(pk_v[pl.ds(base + KE + d, 16)])
                acc = zero16
                for c0 in range(C):
                    w = pk_v[pl.ds(base + 2 * KE + c0 * KE + d, 16)]
                    acc = acc + plsc.load_gather(h_v, [hb + c0]) * w
                plsc.addupdate_scatter(xe_v, [ei], acc)

            _vloop(KE // 16, 4, one)

        ring(pk3.at[l], NCH3, w3_compute)

        def b3_chunk(clen, base):
            pltpu.sync_copy(b3.at[l, pl.ds(hf * EH + base, clen)],
                            pk_v.at[pl.ds(0, clen)])

            def one(i):
                dd = pl.ds(base + i * 16, 16)
                xe_v[dd] = xe_v[dd] + pk_v[pl.ds(i * 16, 16)]

            _vloop(clen // 16, 4, one)

        def b3_loop(c, carry):
            b3_chunk(STG, c * STG)
            return carry

        lax.fori_loop(0, 9, b3_loop, 0)
        b3_chunk(6272, 9 * STG)
        return carry

    lax.fori_loop(0, LAYERS, layer, 0)

    def zo(i):
        ob_v[pl.ds(i * 16, 16)] = zero16

    _vloop(OUTPAD // 16, 4, zo)

    def out_chunk(c, carry):
        pltpu.sync_copy(pout.at[hf, c], pk_v.at[pl.ds(0, 2 * KO)])

        def one(i):
            d = i * 16
            xi = _i32(pk_v[pl.ds(d, 16)])
            tg = _i32(pk_v[pl.ds(KO + d, 16)])
            xv = plsc.load_gather(xe_v, [xi])
            plsc.addupdate_scatter(ob_v, [tg], xv * (1.0 / LAYERS))

        _vloop(KO // 16, 4, one)
        return carry

    lax.fori_loop(0, NCHO, out_chunk, 0)

    @pl.when(hf == 1)
    def _():
        pltpu.sync_copy(ob_v, os_sp.at[p])

    plsc.subcore_barrier()

    @pl.when(hf == 0)
    def _():
        pltpu.sync_copy(os_sp.at[p], pk_v.at[pl.ds(0, OUTPAD)])

        def one(i):
            d = pl.ds(i * 16, 16)
            ob_v[d] = ob_v[d] + pk_v[d]

        _vloop(OUTPAD // 16, 4, one)
        pltpu.sync_copy(ob_v, out.at[b])


@functools.cache
def _get_sc_kernel():
    return functools.partial(
        pl.kernel,
        out_type=jax.ShapeDtypeStruct((B, OUTPAD), jnp.float32),
        mesh=plsc.VectorSubcoreMesh(core_axis_name="c", subcore_axis_name="s"),
        compiler_params=pltpu.CompilerParams(needs_layout_passes=False),
        scratch_types=[
            pltpu.VMEM((EPAD,), jnp.float32),
            pltpu.VMEM((FCPAD,), jnp.float32),
            pltpu.VMEM((2 * PKW,), jnp.float32),
            pltpu.VMEM((OUTPAD,), jnp.float32),
            pltpu.VMEM_SHARED((8, FCPAD), jnp.float32),
            pltpu.VMEM_SHARED((8, OUTPAD), jnp.float32),
            pltpu.SemaphoreType.DMA,
            pltpu.SemaphoreType.DMA,
        ],
    )(_sc_body)


def _pack(w, s0, s1, nch, spk):
    h0 = jnp.pad(w[:, : s0 * C], ((0, 0), (0, nch * KE * C - s0 * C)))
    h1 = jnp.pad(w[:, s0 * C:], ((0, 0), (0, nch * KE * C - s1 * C)))
    wh = jnp.stack([h0, h1], axis=1).reshape(LAYERS, 2, nch, KE, C)
    wh = jnp.swapaxes(wh, -1, -2)
    wf = wh.reshape(LAYERS, 2, nch, 4 * KE)
    sb = jnp.broadcast_to(spk[None], (LAYERS, 2, nch, 2 * KE))
    return jnp.concatenate([sb, wf], axis=-1)


@jax.jit
def kernel(x, Wa1, ba1, Wa2, ba2, W1, B1, W3, B3):
    spad, xz = _tc_state(x[:, :, 0], Wa1, ba1, Wa2, ba2)
    pk1 = _pack(W1, S1[0], S1[1], NCH1, _SPK1)
    pk3 = _pack(W3, S3[0], S3[1], NCH3, _SPK3)
    b1p = jnp.pad(B1, ((0, 0), (0, FCPAD - FC)))
    out = _get_sc_kernel()(xz, spad, pk1, b1p, pk3, B3, _SRCH, _POUT)
    return out[:, :N_OUT]


# speedup vs baseline: 13.4978x; 1.0001x over previous
import functools

import jax
import jax.numpy as jnp
import numpy as np
from jax import lax
from jax.experimental import pallas as pl
from jax.experimental.pallas import tpu as pltpu
from jax.experimental.pallas import tpu_sc as plsc

N = 10000
N_IN = 4000
N_FN = 5500
N_OUT = 500
E = 160000
C = 4
LAYERS = 4
B = 16
LATENT = 100
FC = N_FN * C

EH = E // 2
NP = 10240
EPAD = EH + 16
FCPAD = 22016
OUTPAD = 512
KE = 1024
PKW = 6 * KE
STG = 8192

_rng = np.random.RandomState(0)
_SRC = _rng.randint(0, N_IN + N_FN, size=E).astype(np.int64)
_DST = _rng.randint(N_IN, N, size=E).astype(np.int64)

_e1 = np.nonzero(_DST < N_IN + N_FN)[0]
_f1 = _DST[_e1] - N_IN
_e3 = np.nonzero(_SRC >= N_IN)[0]
_f3 = _SRC[_e3] - N_IN
_eo = np.nonzero(_DST >= N_IN + N_FN)[0]
_do = _DST[_eo] - (N_IN + N_FN)

NNZ1 = _e1.size * C
NNZ3 = _e3.size * C

_i1 = int(np.searchsorted(_e1, EH))
_i3 = int(np.searchsorted(_e3, EH))
_io = int(np.searchsorted(_eo, EH))

S1 = (_i1, _e1.size - _i1)
S3 = (_i3, _e3.size - _i3)
SO = (_io, _eo.size - _io)

NCH1 = 2 * (-(-max(S1) // (2 * KE)))
NCH3 = 2 * (-(-max(S3) // (2 * KE)))
KO = 2048
NCHO = -(-max(SO) // KO)


def _pad_to(a, n, fill):
    out = np.full((n,), fill, dtype=np.int32)
    out[: a.size] = a.astype(np.int32)
    return out


def _build_streams():
    spk1 = np.zeros((2, NCH1, 2 * KE), np.int32)
    spk3 = np.zeros((2, NCH3, 2 * KE), np.int32)
    pout = np.zeros((2, NCHO, 2 * KO), np.int32)
    srch = np.zeros((2, EH), np.int32)
    for h in range(2):
        e1h = _e1[_i1:] - EH if h else _e1[:_i1]
        f1h = _f1[_i1:] if h else _f1[:_i1]
        e3h = _e3[_i3:] - EH if h else _e3[:_i3]
        f3h = _f3[_i3:] if h else _f3[:_i3]
        eoh = _eo[_io:] - EH if h else _eo[:_io]
        doh = _do[_io:] if h else _do[:_io]
        xi = _pad_to(e1h, NCH1 * KE, EH).reshape(NCH1, KE)
        tg = _pad_to(f1h * C, NCH1 * KE, FC).reshape(NCH1, KE)
        spk1[h] = np.concatenate([xi, tg], axis=1)
        hb = _pad_to(f3h * C, NCH3 * KE, FC).reshape(NCH3, KE)
        ei = _pad_to(e3h, NCH3 * KE, EH).reshape(NCH3, KE)
        spk3[h] = np.concatenate([hb, ei], axis=1)
        oi = _pad_to(eoh, NCHO * KO, EH).reshape(NCHO, KO)
        ot = _pad_to(doh, NCHO * KO, OUTPAD - 1).reshape(NCHO, KO)
        pout[h] = np.concatenate([oi, ot], axis=1)
        srch[h] = _SRC[h * EH:(h + 1) * EH].astype(np.int32)
    return (spk1.view(np.float32), spk3.view(np.float32),
            pout.view(np.float32), srch.view(np.float32))


_SPK1, _SPK3, _POUT, _SRCH = (jnp.asarray(a) for a in _build_streams())


def _tc_body(x_ref, wa1_ref, ba1_ref, wa2_ref, ba2_ref, s_ref, xz_ref):
    x = x_ref[...]
    xom = x[:, :N_IN]
    h = xom @ wa1_ref[...] + ba1_ref[...]
    h = jnp.where(h > 0, h, jnp.exp(jnp.minimum(h, 0.0)) - 1.0)
    o = h @ wa2_ref[...] + ba2_ref[...]
    mu = jnp.mean(o, axis=1, keepdims=True)
    var = jnp.mean((o - mu) ** 2, axis=1, keepdims=True)
    s = jax.nn.sigmoid((o - mu) * lax.rsqrt(var + 1e-5))
    s_ref[...] = jnp.pad(s, ((0, 0), (0, FCPAD - FC)))
    mask = lax.broadcasted_iota(jnp.int32, (B, N), 1) < N_IN
    xz_ref[...] = jnp.pad(jnp.where(mask, 0.0, x), ((0, 0), (0, NP - N)))


def _tc_state(x2d, Wa1, ba1, Wa2, ba2):
    return pl.pallas_call(
        _tc_body,
        out_shape=[
            jax.ShapeDtypeStruct((B, FCPAD), jnp.float32),
            jax.ShapeDtypeStruct((B, NP), jnp.float32),
        ],
    )(x2d, Wa1, ba1.reshape(1, LATENT), Wa2, ba2.reshape(1, FC))


def _rsqrt(v):
    i = plsc.bitcast(v, jnp.int32)
    i = 0x5F3759DF - lax.shift_right_logical(i, 1)
    y = plsc.bitcast(i, jnp.float32)
    for _ in range(3):
        y = y * (1.5 - 0.5 * v * y * y)
    return y


def _elu(x):
    return jnp.where(x > 0, x, jnp.exp(jnp.minimum(x, 0.0)) - 1.0)


def _vloop(n_vregs, unroll, body):
    assert n_vregs % unroll == 0

    def outer(i, carry):
        for u in range(unroll):
            body(i * unroll + u)
        return carry

    lax.fori_loop(0, n_vregs // unroll, outer, 0)


def _i32(v):
    return plsc.bitcast(v, jnp.int32)


_GDN = lax.GatherDimensionNumbers(
    offset_dims=(), collapsed_slice_dims=(0,), start_index_map=(0,))


def _perm(v, idx):
    return lax.gather(v, idx[:, None], _GDN, slice_sizes=(1,),
                      mode=lax.GatherScatterMode.PROMISE_IN_BOUNDS)


def _sc_body(xz, spad, pk1, b1p, pk3, b3, srch, pout,
             out, xe_v, h_v, pk_v, ob_v, hs_sp, os_sp, sem0, sem1):
    cid = lax.axis_index("c")
    sid = lax.axis_index("s")
    p = sid // 2
    b = cid * 8 + p
    hf = sid % 2
    zero16 = jnp.zeros((16,), jnp.float32)
    i4 = lax.iota(jnp.int32, 16) * 4

    p1 = lax.iota(jnp.int32, 16) ^ 1
    p2 = lax.iota(jnp.int32, 16) ^ 2

    bufs = (pk_v.at[pl.ds(0, PKW)], pk_v.at[pl.ds(PKW, PKW)])
    sems = (sem0, sem1)

    def ring(src3d, nch, compute):
        pltpu.async_copy(src3d.at[hf, 0], bufs[0], sems[0])

        def step(i, carry):
            for u in range(2):
                c = i * 2 + u
                nxt = 1 - u

                @pl.when(c + 1 < nch)
                def _():
                    pltpu.async_copy(src3d.at[hf, c + 1], bufs[nxt], sems[nxt])

                pltpu.make_async_copy(src3d.at[hf, 0], bufs[u], sems[u]).wait()
                compute(u)
            return carry

        lax.fori_loop(0, nch // 2, step, 0)

    pltpu.sync_copy(xz.at[b], h_v.at[pl.ds(0, NP)])

    def xe0_chunk(clen, base):
        pltpu.sync_copy(srch.at[hf, pl.ds(base, clen)], pk_v.at[pl.ds(0, clen)])

        def one(i):
            idx = _i32(pk_v[pl.ds(i * 16, 16)])
            xe_v[pl.ds(base + i * 16, 16)] = plsc.load_gather(h_v, [idx])

        _vloop(clen // 16, 4, one)

    def xe0_loop(c, carry):
        xe0_chunk(STG, c * STG)
        return carry

    lax.fori_loop(0, 9, xe0_loop, 0)
    xe0_chunk(6272, 9 * STG)
    xe_v[pl.ds(EH, 16)] = zero16

    def layer(l, carry):
        def zh(i):
            h_v[pl.ds(i * 16, 16)] = zero16

        _vloop(FCPAD // 16, 8, zh)

        def w1_compute(u):
            base = u * PKW

            def one(j):
                d = j * 16
                xi = _i32(pk_v[pl.ds(base + d, 16)])
                tg = _i32(pk_v[pl.ds(base + KE + d, 16)])
                xv = plsc.load_gather(xe_v, [xi])
                for c0 in range(C):
                    w = pk_v[pl.ds(base + 2 * KE + c0 * KE + d, 16)]
                    plsc.addupdate_scatter(h_v, [tg + c0], xv * w)

            _vloop(KE // 16, 8, one)

        ring(pk1.at[l], NCH1, w1_compute)

        @pl.when(hf == 0)
        def _():
            pltpu.sync_copy(h_v, hs_sp.at[p])

        plsc.subcore_barrier()

        @pl.when(hf == 1)
        def _():
            def merge_chunk(clen, base):
                pltpu.sync_copy(hs_sp.at[p, pl.ds(base, clen)],
                                pk_v.at[pl.ds(0, clen)])
                pltpu.sync_copy(b1p.at[l, pl.ds(base, clen)],
                                pk_v.at[pl.ds(4096, clen)])

                def one(i):
                    d = i * 16
                    bb = pl.ds(base + d, 16)
                    h_v[bb] = (h_v[bb] + pk_v[pl.ds(d, 16)]
                               + pk_v[pl.ds(4096 + d, 16)])

                _vloop(clen // 16, 4, one)

            def merge_loop(c, carry):
                merge_chunk(4096, c * 4096)
                return carry

            lax.fori_loop(0, 5, merge_loop, 0)
            merge_chunk(1536, 5 * 4096)
            pltpu.sync_copy(h_v, hs_sp.at[p])

        plsc.subcore_barrier()

        @pl.when(hf == 0)
        def _():
            pltpu.sync_copy(hs_sp.at[p], h_v)

        def norm_chunk(clen, base):
            pltpu.sync_copy(spad.at[b, pl.ds(base, clen)], pk_v.at[pl.ds(0, clen)])

            def one(i):
                bb = pl.ds(base + i * 16, 16)
                v = h_v[bb]
                sv = pk_v[pl.ds(i * 16, 16)]
                t = v + _perm(v, p1)
                gs = t + _perm(t, p2)
                mu = gs * 0.25
                d = v - mu
                sq = d * d
                q = sq + _perm(sq, p1)
                var = (q + _perm(q, p2)) * 0.25
                r = _rsqrt(var + 1e-5)
                h_v[bb] = _elu(sv * (d * r))

            _vloop(clen // 16, 4, one)

        def norm_loop(cs, carry):
            norm_chunk(STG, cs * STG)
            return carry

        lax.fori_loop(0, 2, norm_loop, 0)
        norm_chunk(5632, 2 * STG)

        def w3_compute(u):
            base = u * PKW

            def one(j):
                d = j * 16
                hb = _i32(pk_v[pl.ds(base + d, 16)])
                ei = _i32(pk_v[pl.ds(base + KE + d, 16)])
                acc = zero16
                for c0 in range(C):
                    w = pk_v[pl.ds(base + 2 * KE + c0 * KE + d, 16)]
                    acc = acc + plsc.load_gather(h_v, [hb + c0]) * w
                plsc.addupdate_scatter(xe_v, [ei], acc)

            _vloop(KE // 16, 8, one)

        ring(pk3.at[l], NCH3, w3_compute)

        def b3_chunk(clen, base):
            pltpu.sync_copy(b3.at[l, pl.ds(hf * EH + base, clen)],
                            pk_v.at[pl.ds(0, clen)])

            def one(i):
                dd = pl.ds(base + i * 16, 16)
                xe_v[dd] = xe_v[dd] + pk_v[pl.ds(i * 16, 16)]

            _vloop(clen // 16, 4, one)

        def b3_loop(c, carry):
            b3_chunk(STG, c * STG)
            return carry

        lax.fori_loop(0, 9, b3_loop, 0)
        b3_chunk(6272, 9 * STG)
        return carry

    lax.fori_loop(0, LAYERS, layer, 0)

    def zo(i):
        ob_v[pl.ds(i * 16, 16)] = zero16

    _vloop(OUTPAD // 16, 4, zo)

    def out_chunk(c, carry):
        pltpu.sync_copy(pout.at[hf, c], pk_v.at[pl.ds(0, 2 * KO)])

        def one(i):
            d = i * 16
            xi = _i32(pk_v[pl.ds(d, 16)])
            tg = _i32(pk_v[pl.ds(KO + d, 16)])
            xv = plsc.load_gather(xe_v, [xi])
            plsc.addupdate_scatter(ob_v, [tg], xv * (1.0 / LAYERS))

        _vloop(KO // 16, 4, one)
        return carry

    lax.fori_loop(0, NCHO, out_chunk, 0)

    @pl.when(hf == 1)
    def _():
        pltpu.sync_copy(ob_v, os_sp.at[p])

    plsc.subcore_barrier()

    @pl.when(hf == 0)
    def _():
        pltpu.sync_copy(os_sp.at[p], pk_v.at[pl.ds(0, OUTPAD)])

        def one(i):
            d = pl.ds(i * 16, 16)
            ob_v[d] = ob_v[d] + pk_v[d]

        _vloop(OUTPAD // 16, 4, one)
        pltpu.sync_copy(ob_v, out.at[b])


@functools.cache
def _get_sc_kernel():
    return functools.partial(
        pl.kernel,
        out_type=jax.ShapeDtypeStruct((B, OUTPAD), jnp.float32),
        mesh=plsc.VectorSubcoreMesh(core_axis_name="c", subcore_axis_name="s"),
        compiler_params=pltpu.CompilerParams(needs_layout_passes=False),
        scratch_types=[
            pltpu.VMEM((EPAD,), jnp.float32),
            pltpu.VMEM((FCPAD,), jnp.float32),
            pltpu.VMEM((2 * PKW,), jnp.float32),
            pltpu.VMEM((OUTPAD,), jnp.float32),
            pltpu.VMEM_SHARED((8, FCPAD), jnp.float32),
            pltpu.VMEM_SHARED((8, OUTPAD), jnp.float32),
            pltpu.SemaphoreType.DMA,
            pltpu.SemaphoreType.DMA,
        ],
    )(_sc_body)


def _pack(w, s0, s1, nch, spk):
    h0 = jnp.pad(w[:, : s0 * C], ((0, 0), (0, nch * KE * C - s0 * C)))
    h1 = jnp.pad(w[:, s0 * C:], ((0, 0), (0, nch * KE * C - s1 * C)))
    wh = jnp.stack([h0, h1], axis=1).reshape(LAYERS, 2, nch, KE, C)
    wh = jnp.swapaxes(wh, -1, -2)
    wf = wh.reshape(LAYERS, 2, nch, 4 * KE)
    sb = jnp.broadcast_to(spk[None], (LAYERS, 2, nch, 2 * KE))
    return jnp.concatenate([sb, wf], axis=-1)


@jax.jit
def kernel(x, Wa1, ba1, Wa2, ba2, W1, B1, W3, B3):
    spad, xz = _tc_state(x[:, :, 0], Wa1, ba1, Wa2, ba2)
    pk1 = _pack(W1, S1[0], S1[1], NCH1, _SPK1)
    pk3 = _pack(W3, S3[0], S3[1], NCH3, _SPK3)
    b1p = jnp.pad(B1, ((0, 0), (0, FCPAD - FC)))
    out = _get_sc_kernel()(xz, spad, pk1, b1p, pk3, B3, _SRCH, _POUT)
    return out[:, :N_OUT]
